# baseline (speedup 1.0000x reference)
import jax
import jax.numpy as jnp
from jax.experimental import pallas as pl
from jax.experimental.pallas import tpu as pltpu

_E = 64
_ROWS = 512


def _fused_body(x_ref, w_ref, rb_ref, eb_ref, out_ref):
    x = x_ref[...]
    logits = jax.lax.dot_general(
        x, w_ref[...], (((1,), (1,)), ((), ())),
        preferred_element_type=jnp.float32)
    logits = logits + rb_ref[...]

    ml1 = jnp.max(logits, axis=-1, keepdims=True)
    lm = jnp.where(logits == ml1, -jnp.inf, logits)
    ml2 = jnp.max(lm, axis=-1, keepdims=True)

    ex = jnp.exp(logits - ml1)
    r = 1.0 / jnp.sum(ex, axis=-1, keepdims=True)

    s_masked = jnp.where(logits >= ml2, ex, 0.0) * r
    wsum = (1.0 + jnp.exp(ml2 - ml1)) * r

    comb = jnp.dot(s_masked.astype(jnp.bfloat16), eb_ref[...],
                   preferred_element_type=jnp.float32)
    out_ref[...] = wsum * x + comb


def kernel(hidden_states, router_weight, router_bias, expert_bias):
    B, S, H = hidden_states.shape
    BS = B * S
    flat = hidden_states.reshape(BS, H)
    rb = router_bias.reshape(1, _E)
    eb16 = expert_bias.astype(jnp.bfloat16)

    out = pl.pallas_call(
        _fused_body,
        grid=(BS // _ROWS,),
        in_specs=[
            pl.BlockSpec((_ROWS, H), lambda i: (i, 0)),
            pl.BlockSpec((_E, H), lambda i: (0, 0)),
            pl.BlockSpec((1, _E), lambda i: (0, 0)),
            pl.BlockSpec((_E, H), lambda i: (0, 0)),
        ],
        out_specs=pl.BlockSpec((_ROWS, H), lambda i: (i, 0)),
        out_shape=jax.ShapeDtypeStruct((BS, H), jnp.float32),
        compiler_params=pltpu.CompilerParams(
            dimension_semantics=("parallel",)),
    )(flat, router_weight, rb, eb16)
    return out.reshape(B, S, H)


# speedup vs baseline: 1.0760x; 1.0760x over previous
import jax
import jax.numpy as jnp
from jax.experimental import pallas as pl
from jax.experimental.pallas import tpu as pltpu

_E = 64
_ROWS = 1024


def _fused_body(x_ref, w_ref, rb_ref, eb_ref, out_ref):
    x = x_ref[...]
    logits = jax.lax.dot_general(
        x, w_ref[...], (((1,), (1,)), ((), ())),
        preferred_element_type=jnp.float32)
    logits = logits + rb_ref[...]

    ml1 = jnp.max(logits, axis=-1, keepdims=True)
    lm = jnp.where(logits == ml1, -jnp.inf, logits)
    ml2 = jnp.max(lm, axis=-1, keepdims=True)

    ex = jnp.exp(logits - ml1)
    r = 1.0 / jnp.sum(ex, axis=-1, keepdims=True)

    s_masked = jnp.where(logits >= ml2, ex, 0.0) * r
    wsum = (1.0 + jnp.exp(ml2 - ml1)) * r

    comb = jnp.dot(s_masked.astype(jnp.bfloat16), eb_ref[...],
                   preferred_element_type=jnp.float32)
    out_ref[...] = wsum * x + comb


def kernel(hidden_states, router_weight, router_bias, expert_bias):
    B, S, H = hidden_states.shape
    BS = B * S
    flat = hidden_states.reshape(BS, H)
    rb = router_bias.reshape(1, _E)
    eb16 = expert_bias.astype(jnp.bfloat16)

    out = pl.pallas_call(
        _fused_body,
        grid=(BS // _ROWS,),
        in_specs=[
            pl.BlockSpec((_ROWS, H), lambda i: (i, 0)),
            pl.BlockSpec((_E, H), lambda i: (0, 0)),
            pl.BlockSpec((1, _E), lambda i: (0, 0)),
            pl.BlockSpec((_E, H), lambda i: (0, 0)),
        ],
        out_specs=pl.BlockSpec((_ROWS, H), lambda i: (i, 0)),
        out_shape=jax.ShapeDtypeStruct((BS, H), jnp.float32),
        compiler_params=pltpu.CompilerParams(
            dimension_semantics=("parallel",)),
    )(flat, router_weight, rb, eb16)
    return out.reshape(B, S, H)


# speedup vs baseline: 1.0802x; 1.0039x over previous
import jax
import jax.numpy as jnp
from jax.experimental import pallas as pl
from jax.experimental.pallas import tpu as pltpu

_E = 64
_ROWS = 1024


def _fused_body(x_ref, w_ref, rb_ref, eb_ref, out_ref):
    x = x_ref[...]
    logits = jax.lax.dot_general(
        x, w_ref[...], (((1,), (1,)), ((), ())),
        preferred_element_type=jnp.float32)
    logits = logits + rb_ref[...]

    ml1 = jnp.max(logits, axis=-1, keepdims=True)
    lm = jnp.where(logits == ml1, -jnp.inf, logits)
    ml2 = jnp.max(lm, axis=-1, keepdims=True)

    ex = jnp.exp(logits - ml1)
    r = 1.0 / jnp.sum(ex, axis=-1, keepdims=True)

    s_masked = jnp.where(logits >= ml2, ex, 0.0) * r
    wsum = (1.0 + jnp.exp(ml2 - ml1)) * r

    comb = jnp.dot(s_masked.astype(jnp.bfloat16), eb_ref[...],
                   preferred_element_type=jnp.float32)
    out_ref[...] = wsum * x + comb


def kernel(hidden_states, router_weight, router_bias, expert_bias):
    B, S, H = hidden_states.shape
    BS = B * S
    flat = hidden_states.reshape(BS, H)
    rb = router_bias.reshape(1, _E)
    eb16 = expert_bias.astype(jnp.bfloat16)

    out = pl.pallas_call(
        _fused_body,
        grid=(BS // _ROWS,),
        in_specs=[
            pl.BlockSpec((_ROWS, H), lambda i: (i, 0)),
            pl.BlockSpec((_E, H), lambda i: (0, 0)),
            pl.BlockSpec((1, _E), lambda i: (0, 0)),
            pl.BlockSpec((_E, H), lambda i: (0, 0)),
        ],
        out_specs=pl.BlockSpec((_ROWS, H), lambda i: (i, 0)),
        out_shape=jax.ShapeDtypeStruct((BS, H), jnp.float32),
        compiler_params=pltpu.CompilerParams(
            dimension_semantics=("parallel",),
            allow_input_fusion=[False, False, False, True]),
    )(flat, router_weight, rb, eb16)
    return out.reshape(B, S, H)
